# Initial kernel scaffold; baseline (speedup 1.0000x reference)
import jax
import jax.numpy as jnp
from jax.experimental import pallas as pl


def kernel(x, edge_index, edge_attr, batch, W_emb, b_emb, W_conv, b_conv, W_post, b_post, W_f1, b_f1, W_f2, b_f2, W_f3, b_f3):
    raise NotImplementedError("write your pallas kernel here")


# speedup vs baseline: 5.8719x; 5.8719x over previous
import functools

import jax
import jax.numpy as jnp
from jax import lax
from jax.experimental import pallas as pl
from jax.experimental.pallas import tpu as pltpu
from jax.experimental.pallas import tpu_sc as plsc

N_NODES = 10000
N_EDGES = 320000
D_IN = 128
D_EDGE = 16
D_NODE = 18
H_FEA = 16
G_POOL = 64

DP = 32
CNT = D_NODE

NC = 2
NS = 16
NW = NC * NS
EPW = N_EDGES // NW
CH = 400
NCHUNK = EPW // CH
NP = 10240
RPS = NP // NS


def _node_tables_body(x_ref, w_ref, a_ref, b_ref):
    ab = jnp.dot(x_ref[...], w_ref[...], preferred_element_type=jnp.float32)
    a_ref[...] = ab[:, :DP]
    b_ref[...] = ab[:, DP:]


def _node_tables(x, wab):
    return pl.pallas_call(
        _node_tables_body,
        out_shape=(
            jax.ShapeDtypeStruct((NP, DP), jnp.float32),
            jax.ShapeDtypeStruct((NP, DP), jnp.float32),
        ),
    )(x, wab)


_EBLK = 20000


def _edge_table_body(ea_ref, w_ref, bias_ref, c_ref):
    c_ref[...] = (
        jnp.dot(ea_ref[...], w_ref[...], preferred_element_type=jnp.float32)
        + bias_ref[...]
    )


def _edge_table(edge_attr, w3p, cbias):
    grid = N_EDGES // _EBLK
    return pl.pallas_call(
        _edge_table_body,
        grid=(grid,),
        in_specs=[
            pl.BlockSpec((_EBLK, D_EDGE), lambda i: (i, 0)),
            pl.BlockSpec((D_EDGE, DP), lambda i: (0, 0)),
            pl.BlockSpec((1, DP), lambda i: (0, 0)),
        ],
        out_specs=pl.BlockSpec((_EBLK, DP), lambda i: (i, 0)),
        out_shape=jax.ShapeDtypeStruct((N_EDGES, DP), jnp.float32),
    )(edge_attr, w3p, cbias)


def _edge_agg_body(dst_hbm, src_hbm, a_hbm, b_hbm, c_hbm, out_hbm,
                   dst_v, src_v, a_v, b_v, c_v, z_v, acc_sh, sem_a, sem_b):
    cid = lax.axis_index("c")
    sid = lax.axis_index("s")
    wid = cid * NS + sid

    def zrow(j, carry):
        z_v[j, pl.ds(0, 16)] = jnp.zeros((16,), jnp.float32)
        z_v[j, pl.ds(16, 16)] = jnp.zeros((16,), jnp.float32)
        return carry

    lax.fori_loop(0, RPS, zrow, 0)
    pltpu.sync_copy(z_v, acc_sh.at[pl.ds(sid * RPS, RPS)])
    plsc.subcore_barrier()

    def chunk(ci, carry):
        base = wid * EPW + ci * CH
        pltpu.sync_copy(dst_hbm.at[pl.ds(base, CH)], dst_v)
        pltpu.sync_copy(src_hbm.at[pl.ds(base, CH)], src_v)
        cpa = pltpu.async_copy(a_hbm.at[dst_v], a_v, sem_a)
        cpb = pltpu.async_copy(b_hbm.at[src_v], b_v, sem_b)
        pltpu.sync_copy(c_hbm.at[pl.ds(base, CH)], c_v)
        cpa.wait()
        cpb.wait()

        def mrow(j, inner):
            for k in range(DP // 16):
                sl = pl.ds(k * 16, 16)
                m = a_v[j, sl] + b_v[j, sl] + c_v[j, sl]
                c_v[j, sl] = jnp.maximum(m, 0.0)
            return inner

        lax.fori_loop(0, CH, mrow, 0)
        pltpu.sync_copy(c_v, acc_sh.at[dst_v], add=True)
        return carry

    lax.fori_loop(0, NCHUNK, chunk, 0)
    plsc.subcore_barrier()

    pltpu.sync_copy(acc_sh.at[pl.ds(sid * RPS, RPS)], z_v)
    pltpu.sync_copy(z_v, out_hbm.at[cid, pl.ds(sid * RPS, RPS)])


def _edge_agg(dst, src, a, b, c):
    mesh = plsc.VectorSubcoreMesh(
        core_axis_name="c", subcore_axis_name="s",
        num_cores=NC, num_subcores=NS,
    )
    f = functools.partial(
        pl.kernel,
        out_type=jax.ShapeDtypeStruct((NC, NP, DP), jnp.float32),
        mesh=mesh,
        scratch_types=[
            pltpu.VMEM((CH,), jnp.int32),
            pltpu.VMEM((CH,), jnp.int32),
            pltpu.VMEM((CH, DP), jnp.float32),
            pltpu.VMEM((CH, DP), jnp.float32),
            pltpu.VMEM((CH, DP), jnp.float32),
            pltpu.VMEM((RPS, DP), jnp.float32),
            pltpu.VMEM_SHARED((NP, DP), jnp.float32),
            pltpu.SemaphoreType.DMA,
            pltpu.SemaphoreType.DMA,
        ],
        compiler_params=pltpu.CompilerParams(use_tc_tiling_on_sc=False),
    )(_edge_agg_body)
    return f(dst, src, a, b, c)


def _tail_body(p_ref, batch_ref, wpT_ref, bp_ref, w1T_ref, b1_ref,
               w2T_ref, b2_ref, w3T_ref, b3_ref, out_ref):
    s = p_ref[0, :N_NODES] + p_ref[1, :N_NODES]
    cnt = jnp.maximum(s[:, CNT], 1.0)
    h2 = s[:, :D_NODE] / cnt[:, None]
    hp = jnp.dot(h2, wpT_ref[...], preferred_element_type=jnp.float32) + bp_ref[...]
    h3 = jnp.maximum(hp, 0.0) + jnp.log1p(jnp.exp(-jnp.abs(hp)))
    gids = lax.broadcasted_iota(jnp.int32, (N_NODES, G_POOL), 1)
    onehot = (batch_ref[...][:, None] == gids).astype(jnp.float32)
    psum = lax.dot_general(onehot, h3, (((0,), (0,)), ((), ())),
                           preferred_element_type=jnp.float32)
    pcnt = jnp.maximum(jnp.sum(onehot, axis=0), 1.0)
    pooled = psum / pcnt[:, None]
    o = jnp.maximum(jnp.dot(pooled, w1T_ref[...]) + b1_ref[...], 0.0)
    o = jnp.maximum(jnp.dot(o, w2T_ref[...]) + b2_ref[...], 0.0)
    out_ref[...] = jnp.dot(o, w3T_ref[...]) + b3_ref[...]


def _tail(parts, batch, wpT, bp, w1T, b1, w2T, b2, w3T, b3):
    return pl.pallas_call(
        _tail_body,
        out_shape=jax.ShapeDtypeStruct((G_POOL, 1), jnp.float32),
    )(parts, batch, wpT, bp, w1T, b1, w2T, b2, w3T, b3)


def kernel(x, edge_index, edge_attr, batch, W_emb, b_emb, W_conv, b_conv,
           W_post, b_post, W_f1, b_f1, W_f2, b_f2, W_f3, b_f3):
    W1 = W_conv[:, :D_NODE]
    W2 = W_conv[:, D_NODE:2 * D_NODE]
    W3 = W_conv[:, 2 * D_NODE:]
    WA = W1 @ W_emb
    WB = W2 @ W_emb
    bA = W1 @ b_emb
    bB = W2 @ b_emb
    wab = jnp.zeros((D_IN, 2 * DP), jnp.float32)
    wab = wab.at[:, :D_NODE].set(WA.T)
    wab = wab.at[:, DP:DP + D_NODE].set(WB.T)
    w3p = jnp.zeros((D_EDGE, DP), jnp.float32)
    w3p = w3p.at[:, :D_NODE].set(W3.T)
    cbias = jnp.zeros((1, DP), jnp.float32)
    cbias = cbias.at[0, :D_NODE].set(b_conv + bA + bB)
    cbias = cbias.at[0, CNT].set(1.0)

    dst = edge_index[1]
    src = edge_index[0]

    xp = jnp.pad(x, ((0, NP - N_NODES), (0, 0)))
    a_tab, b_tab = _node_tables(xp, wab)
    c_tab = _edge_table(edge_attr, w3p, cbias)
    parts = _edge_agg(dst, src, a_tab, b_tab, c_tab)
    return _tail(parts, batch, W_post.T, b_post, W_f1.T, b_f1,
                 W_f2.T, b_f2, W_f3.T, b_f3)
